# baseline (speedup 1.0000x reference)
import jax
import jax.numpy as jnp
from jax import lax
from jax.experimental import pallas as pl
from jax.experimental.pallas import tpu as pltpu
from jax.experimental.pallas import tpu_sc as plsc

B = 16384
L = 50
D = 32
LANES = 16
NC = 2
NS = 16
NW = NC * NS
BAGS_PER_W = B // NW
C = 16
NCHUNK = BAGS_PER_W // C
ROWS_PER_CHUNK = C * L
GCHUNKS = B // C
VOCAB = 1000000


def _bag_compute(rows_ref, idx_ref, out_ref, j):
    base = j * L
    acc0 = jnp.zeros((LANES,), jnp.float32)
    acc1 = jnp.zeros((LANES,), jnp.float32)
    for r in range(L):
        acc0 = acc0 + rows_ref[base + r, pl.ds(0, LANES)]
        acc1 = acc1 + rows_ref[base + r, pl.ds(LANES, LANES)]
    cnt = jnp.zeros((LANES,), jnp.int32)
    for off in (0, LANES, 2 * LANES):
        idx_v = idx_ref[pl.ds(base + off, LANES)]
        cnt = cnt + plsc.all_reduce_population_count(idx_v != 0)
    tail = idx_ref[pl.ds(base + L - LANES, LANES)]
    lane = lax.iota(jnp.int32, LANES)
    cnt = cnt + plsc.all_reduce_population_count((tail != 0) & (lane >= 14))
    denom = jnp.maximum(cnt.astype(jnp.float32), 1.0)
    out_ref[j, pl.ds(0, LANES)] = acc0 / denom
    out_ref[j, pl.ds(LANES, LANES)] = acc1 / denom


def _emb_bag_kernel(idx_rows, table, out,
                    idxf0, idxf1, rows0, rows1,
                    outb0, outb1, gsem0, gsem1, osem0, osem1):
    wid = lax.axis_index("s") * NC + lax.axis_index("c")
    w_chunk0 = wid * NCHUNK
    w_bag0 = wid * BAGS_PER_W

    idxf = (idxf0, idxf1)
    rows = (rows0, rows1)
    outb = (outb0, outb1)
    gsem = (gsem0, gsem1)
    osem = (osem0, osem1)

    def load_chunk(chunk, b):
        pltpu.sync_copy(idx_rows.at[w_chunk0 + chunk], idxf[b])
        pltpu.async_copy(table.at[idxf[b]], rows[b], gsem[b])

    for b in range(2):
        load_chunk(b, b)

    @pl.loop(0, NCHUNK, step=2)
    def _chunks(g):
        for b in range(2):
            chunk = g + b
            bag0 = w_bag0 + chunk * C
            pltpu.make_async_copy(table.at[idxf[b]], rows[b], gsem[b]).wait()

            @pl.when(chunk >= 2)
            def _():
                pltpu.make_async_copy(
                    outb[b], out.at[pl.ds(bag0, C)], osem[b]).wait()

            @pl.loop(0, C)
            def _bags(j):
                _bag_compute(rows[b], idxf[b], outb[b], j)

            pltpu.async_copy(outb[b], out.at[pl.ds(bag0, C)], osem[b])

            @pl.when(chunk + 2 < NCHUNK)
            def _():
                load_chunk(chunk + 2, b)

    for b in range(2):
        pltpu.make_async_copy(
            outb[b], out.at[pl.ds(0, C)], osem[b]).wait()


@jax.jit
def _emb_bag(idx_rows, table):
    mesh = plsc.VectorSubcoreMesh(core_axis_name="c", subcore_axis_name="s")
    return pl.kernel(
        _emb_bag_kernel,
        out_type=jax.ShapeDtypeStruct((B, D), jnp.float32),
        mesh=mesh,
        compiler_params=pltpu.CompilerParams(
            needs_layout_passes=False, use_tc_tiling_on_sc=False),
        scratch_types=[
            pltpu.VMEM((ROWS_PER_CHUNK,), jnp.int32),
            pltpu.VMEM((ROWS_PER_CHUNK,), jnp.int32),
            pltpu.VMEM((ROWS_PER_CHUNK, D), jnp.float32),
            pltpu.VMEM((ROWS_PER_CHUNK, D), jnp.float32),
            pltpu.VMEM((C, D), jnp.float32),
            pltpu.VMEM((C, D), jnp.float32),
            pltpu.SemaphoreType.DMA,
            pltpu.SemaphoreType.DMA,
            pltpu.SemaphoreType.DMA,
            pltpu.SemaphoreType.DMA,
        ],
    )(idx_rows, table)


def kernel(sentence, weight):
    idx_rows = sentence.astype(jnp.int32).reshape(GCHUNKS, ROWS_PER_CHUNK)
    w32 = jnp.concatenate(
        [weight[0::4], weight[1::4], weight[2::4], weight[3::4]], axis=1
    ).reshape(VOCAB, D)
    return _emb_bag(idx_rows, w32)


# speedup vs baseline: 1.4110x; 1.4110x over previous
import jax
import jax.numpy as jnp
from jax import lax
from jax.experimental import pallas as pl
from jax.experimental.pallas import tpu as pltpu
from jax.experimental.pallas import tpu_sc as plsc

B = 16384
L = 50
D = 32
LANES = 16
NC = 2
NS = 16
NW = NC * NS
BAGS_PER_W = B // NW
C = 16
NCHUNK = BAGS_PER_W // C
ROWS_PER_CHUNK = C * L
GCHUNKS = B // C
VOCAB = 1000000


def _bag_compute(rows_ref, idx_ref, out_ref, j):
    base = j * L
    acc0 = jnp.zeros((LANES,), jnp.float32)
    acc1 = jnp.zeros((LANES,), jnp.float32)
    for r in range(L):
        acc0 = acc0 + rows_ref[base + r, pl.ds(0, LANES)]
        acc1 = acc1 + rows_ref[base + r, pl.ds(LANES, LANES)]
    cnt = jnp.zeros((LANES,), jnp.int32)
    for off in (0, LANES, 2 * LANES):
        idx_v = idx_ref[pl.ds(base + off, LANES)]
        cnt = cnt + plsc.all_reduce_population_count(idx_v != 0)
    tail = idx_ref[pl.ds(base + L - LANES, LANES)]
    lane = lax.iota(jnp.int32, LANES)
    cnt = cnt + plsc.all_reduce_population_count((tail != 0) & (lane >= 14))
    denom = jnp.maximum(cnt.astype(jnp.float32), 1.0)
    out_ref[j, pl.ds(0, LANES)] = acc0 / denom
    out_ref[j, pl.ds(LANES, LANES)] = acc1 / denom


VB = 64
NBLK = VOCAB // VB
BLK_PER_W = -(-NBLK // NW)


def _transpose_kernel(wt, tableA, in0, in1, outb0, outb1,
                      isem0, isem1, osem0, osem1):
    wid = lax.axis_index("s") * NC + lax.axis_index("c")
    inb = (in0, in1)
    outb = (outb0, outb1)
    isem = (isem0, isem1)
    osem = (osem0, osem1)
    lane = lax.iota(jnp.int32, LANES)

    def start_in(it, b):
        blk = wid + it * NW

        @pl.when(blk < NBLK)
        def _():
            pltpu.async_copy(wt.at[:, pl.ds(blk * VB, VB)], inb[b], isem[b])

    for b in range(2):
        start_in(b, b)

    @pl.loop(0, BLK_PER_W, step=2)
    def _blocks(g):
        for b in range(2):
            it = g + b
            blk = wid + it * NW

            @pl.when(blk < NBLK)
            def _():
                pltpu.make_async_copy(
                    wt.at[:, pl.ds(blk * VB, VB)], inb[b], isem[b]).wait()

                @pl.when(it >= 2)
                def _():
                    pltpu.make_async_copy(
                        outb[b], tableA.at[pl.ds(blk * VB, VB)],
                        osem[b]).wait()

                for d in range(D):
                    for gr in range(VB // LANES):
                        v = inb[b][d, pl.ds(gr * LANES, LANES)]
                        plsc.store_scatter(
                            outb[b], [lane + gr * LANES,
                                      jnp.full((LANES,), d, jnp.int32)], v)
                pltpu.async_copy(outb[b], tableA.at[pl.ds(blk * VB, VB)],
                                 osem[b])
                start_in(it + 2, b)

    for b in range(2):
        blk_last = wid + (BLK_PER_W - 2 + b) * NW

        @pl.when(blk_last < NBLK)
        def _():
            pltpu.make_async_copy(
                outb[b], tableA.at[pl.ds(blk_last * VB, VB)], osem[b]).wait()


@jax.jit
def _transpose_table(wt):
    mesh = plsc.VectorSubcoreMesh(core_axis_name="c", subcore_axis_name="s")
    return pl.kernel(
        _transpose_kernel,
        out_type=jax.ShapeDtypeStruct((VOCAB, D), jnp.float32),
        mesh=mesh,
        compiler_params=pltpu.CompilerParams(
            needs_layout_passes=False, use_tc_tiling_on_sc=False),
        scratch_types=[
            pltpu.VMEM((D, VB), jnp.float32),
            pltpu.VMEM((D, VB), jnp.float32),
            pltpu.VMEM((VB, D), jnp.float32),
            pltpu.VMEM((VB, D), jnp.float32),
            pltpu.SemaphoreType.DMA,
            pltpu.SemaphoreType.DMA,
            pltpu.SemaphoreType.DMA,
            pltpu.SemaphoreType.DMA,
        ],
    )(wt)


def _emb_bag_kernel(idx_rows, table, out,
                    idxf0, idxf1, rows0, rows1,
                    outb0, outb1, gsem0, gsem1, osem0, osem1):
    wid = lax.axis_index("s") * NC + lax.axis_index("c")
    w_chunk0 = wid * NCHUNK
    w_bag0 = wid * BAGS_PER_W

    idxf = (idxf0, idxf1)
    rows = (rows0, rows1)
    outb = (outb0, outb1)
    gsem = (gsem0, gsem1)
    osem = (osem0, osem1)

    def load_chunk(chunk, b):
        pltpu.sync_copy(idx_rows.at[w_chunk0 + chunk], idxf[b])
        pltpu.async_copy(table.at[idxf[b]], rows[b], gsem[b])

    for b in range(2):
        load_chunk(b, b)

    @pl.loop(0, NCHUNK, step=2)
    def _chunks(g):
        for b in range(2):
            chunk = g + b
            bag0 = w_bag0 + chunk * C
            pltpu.make_async_copy(table.at[idxf[b]], rows[b], gsem[b]).wait()

            @pl.when(chunk >= 2)
            def _():
                pltpu.make_async_copy(
                    outb[b], out.at[pl.ds(bag0, C)], osem[b]).wait()

            @pl.loop(0, C)
            def _bags(j):
                _bag_compute(rows[b], idxf[b], outb[b], j)

            pltpu.async_copy(outb[b], out.at[pl.ds(bag0, C)], osem[b])

            @pl.when(chunk + 2 < NCHUNK)
            def _():
                load_chunk(chunk + 2, b)

    for b in range(2):
        pltpu.make_async_copy(
            outb[b], out.at[pl.ds(0, C)], osem[b]).wait()


@jax.jit
def _emb_bag(idx_rows, table):
    mesh = plsc.VectorSubcoreMesh(core_axis_name="c", subcore_axis_name="s")
    return pl.kernel(
        _emb_bag_kernel,
        out_type=jax.ShapeDtypeStruct((B, D), jnp.float32),
        mesh=mesh,
        compiler_params=pltpu.CompilerParams(
            needs_layout_passes=False, use_tc_tiling_on_sc=False),
        scratch_types=[
            pltpu.VMEM((ROWS_PER_CHUNK,), jnp.int32),
            pltpu.VMEM((ROWS_PER_CHUNK,), jnp.int32),
            pltpu.VMEM((ROWS_PER_CHUNK, D), jnp.float32),
            pltpu.VMEM((ROWS_PER_CHUNK, D), jnp.float32),
            pltpu.VMEM((C, D), jnp.float32),
            pltpu.VMEM((C, D), jnp.float32),
            pltpu.SemaphoreType.DMA,
            pltpu.SemaphoreType.DMA,
            pltpu.SemaphoreType.DMA,
            pltpu.SemaphoreType.DMA,
        ],
    )(idx_rows, table)


def kernel(sentence, weight):
    idx_rows = sentence.astype(jnp.int32).reshape(GCHUNKS, ROWS_PER_CHUNK)
    tableA = _transpose_table(weight.T)
    return _emb_bag(idx_rows, tableA)


# speedup vs baseline: 4.3861x; 3.1084x over previous
import jax
import jax.numpy as jnp
from jax import lax
from jax.experimental import pallas as pl
from jax.experimental.pallas import tpu as pltpu
from jax.experimental.pallas import tpu_sc as plsc

B = 16384
L = 50
D = 32
LANES = 16
NC = 2
NS = 16
NW = NC * NS
BAGS_PER_W = B // NW
C = 16
NCHUNK = BAGS_PER_W // C
ROWS_PER_CHUNK = C * L
GCHUNKS = B // C
VOCAB = 1000000
DPAD = 40


def _bag_compute(rows_ref, idx_ref, out_ref, j):
    base = j * L
    acc0 = jnp.zeros((LANES,), jnp.float32)
    acc1 = jnp.zeros((LANES,), jnp.float32)
    for r in range(L):
        acc0 = acc0 + rows_ref[base + r, pl.ds(0, LANES)]
        acc1 = acc1 + rows_ref[base + r, pl.ds(LANES, LANES)]
    cnt = jnp.zeros((LANES,), jnp.int32)
    for off in (0, LANES, 2 * LANES):
        idx_v = idx_ref[pl.ds(base + off, LANES)]
        cnt = cnt + plsc.all_reduce_population_count(idx_v != 0)
    tail = idx_ref[pl.ds(base + L - LANES, LANES)]
    lane = lax.iota(jnp.int32, LANES)
    cnt = cnt + plsc.all_reduce_population_count((tail != 0) & (lane >= 14))
    denom = jnp.maximum(cnt.astype(jnp.float32), 1.0)
    out_ref[j, pl.ds(0, LANES)] = acc0 / denom
    out_ref[j, pl.ds(LANES, LANES)] = acc1 / denom


def _emb_bag_kernel(idx_rows, table, out,
                    idxf0, idxf1, rows0, rows1,
                    outb0, outb1, gsem0, gsem1, osem0, osem1):
    wid = lax.axis_index("s") * NC + lax.axis_index("c")
    w_chunk0 = wid * NCHUNK
    w_bag0 = wid * BAGS_PER_W

    idxf = (idxf0, idxf1)
    rows = (rows0, rows1)
    outb = (outb0, outb1)
    gsem = (gsem0, gsem1)
    osem = (osem0, osem1)

    def load_chunk(chunk, b):
        pltpu.sync_copy(idx_rows.at[w_chunk0 + chunk], idxf[b])
        pltpu.async_copy(table.at[idxf[b]], rows[b], gsem[b])

    for b in range(2):
        load_chunk(b, b)

    @pl.loop(0, NCHUNK, step=2)
    def _chunks(g):
        for b in range(2):
            chunk = g + b
            bag0 = w_bag0 + chunk * C
            pltpu.make_async_copy(table.at[idxf[b]], rows[b], gsem[b]).wait()

            @pl.when(chunk >= 2)
            def _():
                pltpu.make_async_copy(
                    outb[b], out.at[pl.ds(bag0, C)], osem[b]).wait()

            @pl.loop(0, C)
            def _bags(j):
                _bag_compute(rows[b], idxf[b], outb[b], j)

            pltpu.async_copy(outb[b], out.at[pl.ds(bag0, C)], osem[b])

            @pl.when(chunk + 2 < NCHUNK)
            def _():
                load_chunk(chunk + 2, b)

    for b in range(2):
        pltpu.make_async_copy(
            outb[b], out.at[pl.ds(0, C)], osem[b]).wait()


@jax.jit
def _emb_bag(idx_rows, table):
    mesh = plsc.VectorSubcoreMesh(core_axis_name="c", subcore_axis_name="s")
    return pl.kernel(
        _emb_bag_kernel,
        out_type=jax.ShapeDtypeStruct((B, D), jnp.float32),
        mesh=mesh,
        compiler_params=pltpu.CompilerParams(
            needs_layout_passes=False, use_tc_tiling_on_sc=False),
        scratch_types=[
            pltpu.VMEM((ROWS_PER_CHUNK,), jnp.int32),
            pltpu.VMEM((ROWS_PER_CHUNK,), jnp.int32),
            pltpu.VMEM((ROWS_PER_CHUNK, DPAD), jnp.float32),
            pltpu.VMEM((ROWS_PER_CHUNK, DPAD), jnp.float32),
            pltpu.VMEM((C, D), jnp.float32),
            pltpu.VMEM((C, D), jnp.float32),
            pltpu.SemaphoreType.DMA,
            pltpu.SemaphoreType.DMA,
            pltpu.SemaphoreType.DMA,
            pltpu.SemaphoreType.DMA,
        ],
    )(idx_rows, table)


def kernel(sentence, weight):
    idx_rows = sentence.astype(jnp.int32).reshape(GCHUNKS, ROWS_PER_CHUNK)
    wpad = jnp.pad(weight, ((0, 0), (0, DPAD - D)))
    return _emb_bag(idx_rows, wpad)


# speedup vs baseline: 8.0181x; 1.8281x over previous
import jax
import jax.numpy as jnp
from jax import lax
from jax.experimental import pallas as pl
from jax.experimental.pallas import tpu as pltpu
from jax.experimental.pallas import tpu_sc as plsc

B = 16384
L = 50
D = 32
LANES = 16
NC = 2
NS = 16
NW = NC * NS
BAGS_PER_W = B // NW
C = 16
NCHUNK = BAGS_PER_W // C
ROWS_PER_CHUNK = C * L
GCHUNKS = B // C
VOCAB = 1000000


def _bag_compute(rows_ref, idx_ref, out_ref, j):
    base = j * L
    acc0 = jnp.zeros((LANES,), jnp.float32)
    acc1 = jnp.zeros((LANES,), jnp.float32)
    for r in range(L):
        acc0 = acc0 + rows_ref[base + r, pl.ds(0, LANES)]
        acc1 = acc1 + rows_ref[base + r, pl.ds(LANES, LANES)]
    cnt = jnp.zeros((LANES,), jnp.int32)
    for off in (0, LANES, 2 * LANES):
        idx_v = idx_ref[pl.ds(base + off, LANES)]
        cnt = cnt + plsc.all_reduce_population_count(idx_v != 0)
    tail = idx_ref[pl.ds(base + L - LANES, LANES)]
    lane = lax.iota(jnp.int32, LANES)
    cnt = cnt + plsc.all_reduce_population_count((tail != 0) & (lane >= 14))
    denom = jnp.maximum(cnt.astype(jnp.float32), 1.0)
    out_ref[j, pl.ds(0, LANES)] = acc0 / denom
    out_ref[j, pl.ds(LANES, LANES)] = acc1 / denom


def _emb_bag_kernel(idx_rows, table, out,
                    idxf0, idxf1, rows0, rows1,
                    outb0, outb1, gsem0, gsem1, osem0, osem1):
    wid = lax.axis_index("s") * NC + lax.axis_index("c")
    w_chunk0 = wid * NCHUNK
    w_bag0 = wid * BAGS_PER_W

    idxf = (idxf0, idxf1)
    rows = (rows0, rows1)
    outb = (outb0, outb1)
    gsem = (gsem0, gsem1)
    osem = (osem0, osem1)

    def load_chunk(chunk, b):
        pltpu.sync_copy(idx_rows.at[w_chunk0 + chunk], idxf[b])
        pltpu.async_copy(table.at[idxf[b]], rows[b], gsem[b])

    for b in range(2):
        load_chunk(b, b)

    @pl.loop(0, NCHUNK, step=2)
    def _chunks(g):
        for b in range(2):
            chunk = g + b
            bag0 = w_bag0 + chunk * C
            pltpu.make_async_copy(table.at[idxf[b]], rows[b], gsem[b]).wait()

            @pl.when(chunk >= 2)
            def _():
                pltpu.make_async_copy(
                    outb[b], out.at[pl.ds(bag0, C)], osem[b]).wait()

            @pl.loop(0, C)
            def _bags(j):
                _bag_compute(rows[b], idxf[b], outb[b], j)

            pltpu.async_copy(outb[b], out.at[pl.ds(bag0, C)], osem[b])

            @pl.when(chunk + 2 < NCHUNK)
            def _():
                load_chunk(chunk + 2, b)

    for b in range(2):
        pltpu.make_async_copy(
            outb[b], out.at[pl.ds(0, C)], osem[b]).wait()


@jax.jit
def _emb_bag(idx_rows, table):
    mesh = plsc.VectorSubcoreMesh(core_axis_name="c", subcore_axis_name="s")
    return pl.kernel(
        _emb_bag_kernel,
        out_type=jax.ShapeDtypeStruct((B, D), jnp.float32),
        mesh=mesh,
        compiler_params=pltpu.CompilerParams(
            needs_layout_passes=False, use_tc_tiling_on_sc=False),
        scratch_types=[
            pltpu.VMEM((ROWS_PER_CHUNK,), jnp.int32),
            pltpu.VMEM((ROWS_PER_CHUNK,), jnp.int32),
            pltpu.VMEM((ROWS_PER_CHUNK, D), jnp.float32),
            pltpu.VMEM((ROWS_PER_CHUNK, D), jnp.float32),
            pltpu.VMEM((C, D), jnp.float32),
            pltpu.VMEM((C, D), jnp.float32),
            pltpu.SemaphoreType.DMA,
            pltpu.SemaphoreType.DMA,
            pltpu.SemaphoreType.DMA,
            pltpu.SemaphoreType.DMA,
        ],
    )(idx_rows, table)


def kernel(sentence, weight):
    idx_rows = (sentence.astype(jnp.int32) * 4).reshape(GCHUNKS, ROWS_PER_CHUNK)
    wpad = jnp.pad(weight, ((0, 0), (0, 128 - D)))
    w4 = wpad.reshape(4 * VOCAB, D)
    return _emb_bag(idx_rows, w4)
